# baseline (device time: 28733 ns/iter reference)
import jax
import jax.numpy as jnp
from jax import lax
from jax.experimental import pallas as pl
from jax.experimental.pallas import tpu as pltpu

N_DEV = 32


def kernel(x, w_mat):
    m_per, k = x.shape
    n = w_mat.shape[1]
    n_per = n // N_DEV
    m_total = m_per * N_DEV

    def body(x_ref, w_ref, out_ref, y_ref, send_sems, recv_sems):
        my = lax.axis_index("i")

        barrier = pltpu.get_barrier_semaphore()
        for p in range(N_DEV):
            pl.semaphore_signal(
                barrier, inc=1,
                device_id=(p,), device_id_type=pl.DeviceIdType.MESH,
            )

        y = jnp.dot(x_ref[...], w_ref[...], preferred_element_type=jnp.float32)
        y = y * jax.nn.sigmoid(y)
        y_ref[...] = y.reshape(m_per, N_DEV, n_per).transpose(1, 0, 2)

        pl.semaphore_wait(barrier, N_DEV)

        for k in range(1, N_DEV):
            p = lax.rem(my + k, N_DEV)
            rdma = pltpu.make_async_remote_copy(
                src_ref=y_ref.at[p],
                dst_ref=out_ref.at[pl.ds(my * m_per, m_per), :],
                send_sem=send_sems.at[k],
                recv_sem=recv_sems.at[k],
                device_id=(p,),
                device_id_type=pl.DeviceIdType.MESH,
            )
            rdma.start()

        out_ref[pl.ds(my * m_per, m_per), :] = y_ref[my]

        for k in range(1, N_DEV):
            wait = pltpu.make_async_remote_copy(
                src_ref=y_ref.at[0],
                dst_ref=out_ref.at[pl.ds(0, m_per), :],
                send_sem=send_sems.at[k],
                recv_sem=recv_sems.at[k],
                device_id=(0,),
                device_id_type=pl.DeviceIdType.MESH,
            )
            wait.wait_send()
            wait.wait_recv()

    return pl.pallas_call(
        body,
        out_shape=jax.ShapeDtypeStruct((m_total, n_per), jnp.float32),
        in_specs=[
            pl.BlockSpec(memory_space=pltpu.VMEM),
            pl.BlockSpec(memory_space=pltpu.VMEM),
        ],
        out_specs=pl.BlockSpec(memory_space=pltpu.VMEM),
        scratch_shapes=[
            pltpu.VMEM((N_DEV, m_per, n_per), jnp.float32),
            pltpu.SemaphoreType.DMA((N_DEV,)),
            pltpu.SemaphoreType.DMA((N_DEV,)),
        ],
        compiler_params=pltpu.CompilerParams(collective_id=0),
    )(x, w_mat)
